# baseline (speedup 1.0000x reference)
import functools

import jax
import jax.numpy as jnp
from jax import lax
from jax.experimental import pallas as pl
from jax.experimental.pallas import tpu as pltpu
from jax.experimental.pallas import tpu_sc as plsc

B = 16384
D = 64
NC = 2
NS = 16
NW = NC * NS
BPW = B // NW
CHUNK = 128
NCHUNK = BPW // CHUNK
GROUPS = BPW // 16


def _fm_kernel(user_h, item_h, uwt_h, iwt_h, ubt_h, ibt_h, bias_h, out_h,
               uidx_v, iidx_v, ubuf_v, ibuf_v, ub_v, ib_v, bias_v, out_v,
               sem):
    wid = lax.axis_index("s") * NC + lax.axis_index("c")

    pltpu.sync_copy(user_h.at[wid], uidx_v)
    pltpu.sync_copy(item_h.at[wid], iidx_v)
    pltpu.sync_copy(bias_h, bias_v)

    for j in range(NCHUNK):
        pltpu.async_copy(ubt_h.at[0].at[uidx_v.at[j]],
                         ub_v.at[pl.ds(j * CHUNK, CHUNK)], sem)
        pltpu.async_copy(ibt_h.at[0].at[iidx_v.at[j]],
                         ib_v.at[pl.ds(j * CHUNK, CHUNK)], sem)

    def gather_f(f, carry):
        for j in range(NCHUNK):
            pltpu.async_copy(uwt_h.at[f].at[uidx_v.at[j]],
                             ubuf_v.at[f, pl.ds(j * CHUNK, CHUNK)], sem)
            pltpu.async_copy(iwt_h.at[f].at[iidx_v.at[j]],
                             ibuf_v.at[f, pl.ds(j * CHUNK, CHUNK)], sem)
        for j in range(NCHUNK):
            pltpu.make_async_copy(uwt_h.at[f].at[uidx_v.at[j]],
                                  ubuf_v.at[f, pl.ds(j * CHUNK, CHUNK)],
                                  sem).wait()
            pltpu.make_async_copy(iwt_h.at[f].at[iidx_v.at[j]],
                                  ibuf_v.at[f, pl.ds(j * CHUNK, CHUNK)],
                                  sem).wait()
        return carry

    lax.fori_loop(0, D, gather_f, 0)

    for j in range(NCHUNK):
        pltpu.make_async_copy(ubt_h.at[0].at[uidx_v.at[j]],
                              ub_v.at[pl.ds(j * CHUNK, CHUNK)], sem).wait()
        pltpu.make_async_copy(ibt_h.at[0].at[iidx_v.at[j]],
                              ib_v.at[pl.ds(j * CHUNK, CHUNK)], sem).wait()

    b0 = bias_v[...]

    def group_body(g, carry):
        acc = ub_v[pl.ds(g * 16, 16)] + ib_v[pl.ds(g * 16, 16)] + b0
        for f in range(D):
            gu = ubuf_v[f, pl.ds(g * 16, 16)]
            gi = ibuf_v[f, pl.ds(g * 16, 16)]
            acc = acc + gu * gi
        out_v[pl.ds(g * 16, 16)] = acc
        return carry

    lax.fori_loop(0, GROUPS, group_body, 0)
    pltpu.sync_copy(out_v, out_h.at[pl.ds(wid * BPW, BPW)])


def kernel(user, item, context, embed_user_w, embed_item_w,
           u_bias_w, i_bias_w, bias_):
    del context
    user3 = user.astype(jnp.int32).reshape(NW, NCHUNK, CHUNK)
    item3 = item.astype(jnp.int32).reshape(NW, NCHUNK, CHUNK)
    uw_t = embed_user_w.T
    iw_t = embed_item_w.T
    ub_t = u_bias_w.T
    ib_t = i_bias_w.T
    bias16 = jnp.broadcast_to(bias_.reshape(()), (16,))

    mesh = plsc.VectorSubcoreMesh(core_axis_name="c", subcore_axis_name="s")
    fm = functools.partial(
        pl.kernel,
        out_type=jax.ShapeDtypeStruct((B,), jnp.float32),
        mesh=mesh,
        compiler_params=pltpu.CompilerParams(
            needs_layout_passes=False, use_tc_tiling_on_sc=False),
        scratch_types=[
            pltpu.VMEM((NCHUNK, CHUNK), jnp.int32),
            pltpu.VMEM((NCHUNK, CHUNK), jnp.int32),
            pltpu.VMEM((D, BPW), jnp.float32),
            pltpu.VMEM((D, BPW), jnp.float32),
            pltpu.VMEM((BPW,), jnp.float32),
            pltpu.VMEM((BPW,), jnp.float32),
            pltpu.VMEM((16,), jnp.float32),
            pltpu.VMEM((BPW,), jnp.float32),
            pltpu.SemaphoreType.DMA,
        ],
    )(_fm_kernel)
    return fm(user3, item3, uw_t, iw_t, ub_t, ib_t, bias16)


# speedup vs baseline: 15.9972x; 15.9972x over previous
import functools

import jax
import jax.numpy as jnp
from jax import lax
from jax.experimental import pallas as pl
from jax.experimental.pallas import tpu as pltpu
from jax.experimental.pallas import tpu_sc as plsc

B = 16384
D = 64
NC = 2
NS = 16
NW = NC * NS
BPW = B // NW
CHUNK = 128
NCHUNK = BPW // CHUNK
GROUPS = BPW // 16

V = 1000000
ROWS_PER_W = 2 * D // NW
VMAIN = (V // CHUNK) * CHUNK
VTAIL = V - VMAIN


CCHUNK = 16128
NCH = VMAIN // CCHUNK


def _depad_kernel(uwt_h, iwt_h, uflat_h, iflat_h, buf_v, tail_v, sem):
    wid = lax.axis_index("s") * NC + lax.axis_index("c")

    def row_body(k, carry):
        r = wid * ROWS_PER_W + k
        f = lax.rem(r, D)

        def copy_row(src_h, dst_h):
            def chunk_body(c, carry2):
                pltpu.sync_copy(src_h.at[f].at[pl.ds(c * CCHUNK, CCHUNK)],
                                buf_v)
                pltpu.sync_copy(buf_v,
                                dst_h.at[pl.ds(f * V + c * CCHUNK, CCHUNK)])
                return carry2

            lax.fori_loop(0, NCH, chunk_body, 0)
            pltpu.sync_copy(src_h.at[f].at[pl.ds(VMAIN, VTAIL)], tail_v)
            pltpu.sync_copy(tail_v, dst_h.at[pl.ds(f * V + VMAIN, VTAIL)])

        @pl.when(r < D)
        def _():
            copy_row(uwt_h, uflat_h)

        @pl.when(r >= D)
        def _():
            copy_row(iwt_h, iflat_h)

        return carry

    lax.fori_loop(0, ROWS_PER_W, row_body, 0)


def _fm_kernel(user_h, item_h, uwt_h, iwt_h, ubt_h, ibt_h, bias_h, out_h,
               uidx_v, iidx_v, ubuf_v, ibuf_v, ub_v, ib_v, bias_v, out_v,
               sem):
    wid = lax.axis_index("s") * NC + lax.axis_index("c")

    pltpu.sync_copy(user_h.at[wid], uidx_v)
    pltpu.sync_copy(item_h.at[wid], iidx_v)
    pltpu.sync_copy(bias_h, bias_v)

    for j in range(NCHUNK):
        pltpu.async_copy(ubt_h.at[0].at[uidx_v.at[j]],
                         ub_v.at[pl.ds(j * CHUNK, CHUNK)], sem)
        pltpu.async_copy(ibt_h.at[0].at[iidx_v.at[j]],
                         ib_v.at[pl.ds(j * CHUNK, CHUNK)], sem)

    def gather_f(f, carry):
        for j in range(NCHUNK):
            pltpu.async_copy(uwt_h.at[f].at[uidx_v.at[j]],
                             ubuf_v.at[f, pl.ds(j * CHUNK, CHUNK)], sem)
            pltpu.async_copy(iwt_h.at[f].at[iidx_v.at[j]],
                             ibuf_v.at[f, pl.ds(j * CHUNK, CHUNK)], sem)
        for j in range(NCHUNK):
            pltpu.make_async_copy(uwt_h.at[f].at[uidx_v.at[j]],
                                  ubuf_v.at[f, pl.ds(j * CHUNK, CHUNK)],
                                  sem).wait()
            pltpu.make_async_copy(iwt_h.at[f].at[iidx_v.at[j]],
                                  ibuf_v.at[f, pl.ds(j * CHUNK, CHUNK)],
                                  sem).wait()
        return carry

    lax.fori_loop(0, D, gather_f, 0)

    for j in range(NCHUNK):
        pltpu.make_async_copy(ubt_h.at[0].at[uidx_v.at[j]],
                              ub_v.at[pl.ds(j * CHUNK, CHUNK)], sem).wait()
        pltpu.make_async_copy(ibt_h.at[0].at[iidx_v.at[j]],
                              ib_v.at[pl.ds(j * CHUNK, CHUNK)], sem).wait()

    b0 = bias_v[...]

    def group_body(g, carry):
        acc = ub_v[pl.ds(g * 16, 16)] + ib_v[pl.ds(g * 16, 16)] + b0
        for f in range(D):
            gu = ubuf_v[f, pl.ds(g * 16, 16)]
            gi = ibuf_v[f, pl.ds(g * 16, 16)]
            acc = acc + gu * gi
        out_v[pl.ds(g * 16, 16)] = acc
        return carry

    lax.fori_loop(0, GROUPS, group_body, 0)
    pltpu.sync_copy(out_v, out_h.at[pl.ds(wid * BPW, BPW)])


def kernel(user, item, context, embed_user_w, embed_item_w,
           u_bias_w, i_bias_w, bias_):
    del context
    user3 = user.astype(jnp.int32).reshape(NW, NCHUNK, CHUNK)
    item3 = item.astype(jnp.int32).reshape(NW, NCHUNK, CHUNK)
    uw_t = embed_user_w.T
    iw_t = embed_item_w.T
    ub_t = u_bias_w.T
    ib_t = i_bias_w.T
    bias16 = jnp.broadcast_to(bias_.reshape(()), (16,))

    mesh = plsc.VectorSubcoreMesh(core_axis_name="c", subcore_axis_name="s")

    depad = functools.partial(
        pl.kernel,
        out_type=(jax.ShapeDtypeStruct((D * V,), jnp.float32),
                  jax.ShapeDtypeStruct((D * V,), jnp.float32)),
        mesh=mesh,
        compiler_params=pltpu.CompilerParams(
            needs_layout_passes=False, use_tc_tiling_on_sc=True),
        scratch_types=[
            pltpu.VMEM((CCHUNK,), jnp.float32),
            pltpu.VMEM((VTAIL,), jnp.float32),
            pltpu.SemaphoreType.DMA,
        ],
    )(_depad_kernel)
    uflat, iflat = depad(uw_t, iw_t)

    fm = functools.partial(
        pl.kernel,
        out_type=jax.ShapeDtypeStruct((B,), jnp.float32),
        mesh=mesh,
        compiler_params=pltpu.CompilerParams(
            needs_layout_passes=False, use_tc_tiling_on_sc=False),
        scratch_types=[
            pltpu.VMEM((NCHUNK, CHUNK), jnp.int32),
            pltpu.VMEM((NCHUNK, CHUNK), jnp.int32),
            pltpu.VMEM((D, BPW), jnp.float32),
            pltpu.VMEM((D, BPW), jnp.float32),
            pltpu.VMEM((BPW,), jnp.float32),
            pltpu.VMEM((BPW,), jnp.float32),
            pltpu.VMEM((16,), jnp.float32),
            pltpu.VMEM((BPW,), jnp.float32),
            pltpu.SemaphoreType.DMA,
        ],
    )(_fm_kernel)
    return fm(user3, item3, uflat.reshape(D, V), iflat.reshape(D, V),
              ub_t, ib_t, bias16)


# speedup vs baseline: 21.9875x; 1.3745x over previous
import functools

import jax
import jax.numpy as jnp
from jax import lax
from jax.experimental import pallas as pl
from jax.experimental.pallas import tpu as pltpu
from jax.experimental.pallas import tpu_sc as plsc

B = 16384
D = 64
NC = 2
NS = 16
NW = NC * NS
BPW = B // NW
CHUNK = 128
NCHUNK = BPW // CHUNK
GROUPS = BPW // 16

V = 1000000
ROWS_PER_W = 2 * D // NW
VMAIN = (V // CHUNK) * CHUNK
VTAIL = V - VMAIN


CCHUNK = 16128
NCH = VMAIN // CCHUNK
NSLOT = 4
TOTCH = ROWS_PER_W * NCH


def _depad_kernel(uwt_h, iwt_h, uflat_h, iflat_h,
                  buf0_v, buf1_v, buf2_v, buf3_v, tail_v,
                  rsem, wsem, tsem):
    bufs = [buf0_v, buf1_v, buf2_v, buf3_v]
    wid = lax.axis_index("s") * NC + lax.axis_index("c")

    def src_dst(g):
        k = lax.div(g, NCH)
        c = lax.rem(g, NCH)
        r = wid * ROWS_PER_W + k
        f = lax.rem(r, D)
        return r, f, c * CCHUNK

    def fire_read(slot, g):
        r, f, off = src_dst(g)

        @pl.when(r < D)
        def _():
            pltpu.async_copy(uwt_h.at[f].at[pl.ds(off, CCHUNK)],
                             bufs[slot], rsem)

        @pl.when(r >= D)
        def _():
            pltpu.async_copy(iwt_h.at[f].at[pl.ds(off, CCHUNK)],
                             bufs[slot], rsem)

    def wait_read(slot, g):
        r, f, off = src_dst(g)

        @pl.when(r < D)
        def _():
            pltpu.make_async_copy(uwt_h.at[f].at[pl.ds(off, CCHUNK)],
                                  bufs[slot], rsem).wait()

        @pl.when(r >= D)
        def _():
            pltpu.make_async_copy(iwt_h.at[f].at[pl.ds(off, CCHUNK)],
                                  bufs[slot], rsem).wait()

    def fire_write(slot, g):
        r, f, off = src_dst(g)

        @pl.when(r < D)
        def _():
            pltpu.async_copy(bufs[slot],
                             uflat_h.at[pl.ds(f * V + off, CCHUNK)], wsem)

        @pl.when(r >= D)
        def _():
            pltpu.async_copy(bufs[slot],
                             iflat_h.at[pl.ds(f * V + off, CCHUNK)], wsem)

    def wait_write(slot, g):
        r, f, off = src_dst(g)

        @pl.when(r < D)
        def _():
            pltpu.make_async_copy(bufs[slot],
                                  uflat_h.at[pl.ds(f * V + off, CCHUNK)],
                                  wsem).wait()

        @pl.when(r >= D)
        def _():
            pltpu.make_async_copy(bufs[slot],
                                  iflat_h.at[pl.ds(f * V + off, CCHUNK)],
                                  wsem).wait()

    for slot in range(NSLOT):
        fire_read(slot, jnp.int32(slot))

    def round_body(t, carry):
        for slot in range(NSLOT):
            g = t * NSLOT + slot
            wait_read(slot, g)
            fire_write(slot, g)
        for slot in range(NSLOT):
            g = t * NSLOT + slot
            wait_write(slot, g)

            @pl.when(g + NSLOT < TOTCH)
            def _():
                fire_read(slot, g + NSLOT)

        return carry

    lax.fori_loop(0, TOTCH // NSLOT, round_body, 0)

    def tail_body(k, carry):
        r = wid * ROWS_PER_W + k
        f = lax.rem(r, D)

        @pl.when(r < D)
        def _():
            pltpu.async_copy(uwt_h.at[f].at[pl.ds(VMAIN, VTAIL)],
                             tail_v, tsem)
            pltpu.make_async_copy(uwt_h.at[f].at[pl.ds(VMAIN, VTAIL)],
                                  tail_v, tsem).wait()
            pltpu.async_copy(tail_v,
                             uflat_h.at[pl.ds(f * V + VMAIN, VTAIL)], tsem)
            pltpu.make_async_copy(tail_v,
                                  uflat_h.at[pl.ds(f * V + VMAIN, VTAIL)],
                                  tsem).wait()

        @pl.when(r >= D)
        def _():
            pltpu.async_copy(iwt_h.at[f].at[pl.ds(VMAIN, VTAIL)],
                             tail_v, tsem)
            pltpu.make_async_copy(iwt_h.at[f].at[pl.ds(VMAIN, VTAIL)],
                                  tail_v, tsem).wait()
            pltpu.async_copy(tail_v,
                             iflat_h.at[pl.ds(f * V + VMAIN, VTAIL)], tsem)
            pltpu.make_async_copy(tail_v,
                                  iflat_h.at[pl.ds(f * V + VMAIN, VTAIL)],
                                  tsem).wait()

        return carry

    lax.fori_loop(0, ROWS_PER_W, tail_body, 0)


def _fm_kernel(user_h, item_h, uwt_h, iwt_h, ubt_h, ibt_h, bias_h, out_h,
               uidx_v, iidx_v, ubuf_v, ibuf_v, ub_v, ib_v, bias_v, out_v,
               sem):
    wid = lax.axis_index("s") * NC + lax.axis_index("c")

    pltpu.sync_copy(user_h.at[wid], uidx_v)
    pltpu.sync_copy(item_h.at[wid], iidx_v)
    pltpu.sync_copy(bias_h, bias_v)

    for j in range(NCHUNK):
        pltpu.async_copy(ubt_h.at[0].at[uidx_v.at[j]],
                         ub_v.at[pl.ds(j * CHUNK, CHUNK)], sem)
        pltpu.async_copy(ibt_h.at[0].at[iidx_v.at[j]],
                         ib_v.at[pl.ds(j * CHUNK, CHUNK)], sem)

    def gather_f(f, carry):
        for j in range(NCHUNK):
            pltpu.async_copy(uwt_h.at[f].at[uidx_v.at[j]],
                             ubuf_v.at[f, pl.ds(j * CHUNK, CHUNK)], sem)
            pltpu.async_copy(iwt_h.at[f].at[iidx_v.at[j]],
                             ibuf_v.at[f, pl.ds(j * CHUNK, CHUNK)], sem)
        for j in range(NCHUNK):
            pltpu.make_async_copy(uwt_h.at[f].at[uidx_v.at[j]],
                                  ubuf_v.at[f, pl.ds(j * CHUNK, CHUNK)],
                                  sem).wait()
            pltpu.make_async_copy(iwt_h.at[f].at[iidx_v.at[j]],
                                  ibuf_v.at[f, pl.ds(j * CHUNK, CHUNK)],
                                  sem).wait()
        return carry

    lax.fori_loop(0, D, gather_f, 0)

    for j in range(NCHUNK):
        pltpu.make_async_copy(ubt_h.at[0].at[uidx_v.at[j]],
                              ub_v.at[pl.ds(j * CHUNK, CHUNK)], sem).wait()
        pltpu.make_async_copy(ibt_h.at[0].at[iidx_v.at[j]],
                              ib_v.at[pl.ds(j * CHUNK, CHUNK)], sem).wait()

    b0 = bias_v[...]

    def group_body(g, carry):
        acc = ub_v[pl.ds(g * 16, 16)] + ib_v[pl.ds(g * 16, 16)] + b0
        for f in range(D):
            gu = ubuf_v[f, pl.ds(g * 16, 16)]
            gi = ibuf_v[f, pl.ds(g * 16, 16)]
            acc = acc + gu * gi
        out_v[pl.ds(g * 16, 16)] = acc
        return carry

    lax.fori_loop(0, GROUPS, group_body, 0)
    pltpu.sync_copy(out_v, out_h.at[pl.ds(wid * BPW, BPW)])


def kernel(user, item, context, embed_user_w, embed_item_w,
           u_bias_w, i_bias_w, bias_):
    del context
    user3 = user.astype(jnp.int32).reshape(NW, NCHUNK, CHUNK)
    item3 = item.astype(jnp.int32).reshape(NW, NCHUNK, CHUNK)
    uw_t = embed_user_w.T
    iw_t = embed_item_w.T
    ub_t = u_bias_w.T
    ib_t = i_bias_w.T
    bias16 = jnp.broadcast_to(bias_.reshape(()), (16,))

    mesh = plsc.VectorSubcoreMesh(core_axis_name="c", subcore_axis_name="s")

    depad = functools.partial(
        pl.kernel,
        out_type=(jax.ShapeDtypeStruct((D * V,), jnp.float32),
                  jax.ShapeDtypeStruct((D * V,), jnp.float32)),
        mesh=mesh,
        compiler_params=pltpu.CompilerParams(
            needs_layout_passes=False, use_tc_tiling_on_sc=True),
        scratch_types=[
            pltpu.VMEM((CCHUNK,), jnp.float32),
            pltpu.VMEM((CCHUNK,), jnp.float32),
            pltpu.VMEM((CCHUNK,), jnp.float32),
            pltpu.VMEM((CCHUNK,), jnp.float32),
            pltpu.VMEM((VTAIL,), jnp.float32),
            pltpu.SemaphoreType.DMA,
            pltpu.SemaphoreType.DMA,
            pltpu.SemaphoreType.DMA,
        ],
    )(_depad_kernel)
    uflat, iflat = depad(uw_t, iw_t)

    fm = functools.partial(
        pl.kernel,
        out_type=jax.ShapeDtypeStruct((B,), jnp.float32),
        mesh=mesh,
        compiler_params=pltpu.CompilerParams(
            needs_layout_passes=False, use_tc_tiling_on_sc=False),
        scratch_types=[
            pltpu.VMEM((NCHUNK, CHUNK), jnp.int32),
            pltpu.VMEM((NCHUNK, CHUNK), jnp.int32),
            pltpu.VMEM((D, BPW), jnp.float32),
            pltpu.VMEM((D, BPW), jnp.float32),
            pltpu.VMEM((BPW,), jnp.float32),
            pltpu.VMEM((BPW,), jnp.float32),
            pltpu.VMEM((16,), jnp.float32),
            pltpu.VMEM((BPW,), jnp.float32),
            pltpu.SemaphoreType.DMA,
        ],
    )(_fm_kernel)
    return fm(user3, item3, uflat.reshape(D, V), iflat.reshape(D, V),
              ub_t, ib_t, bias16)


# speedup vs baseline: 22.1253x; 1.0063x over previous
import functools

import jax
import jax.numpy as jnp
from jax import lax
from jax.experimental import pallas as pl
from jax.experimental.pallas import tpu as pltpu
from jax.experimental.pallas import tpu_sc as plsc

B = 16384
D = 64
NC = 2
NS = 16
NW = NC * NS
BPW = B // NW
CHUNK = 128
NCHUNK = BPW // CHUNK
GROUPS = BPW // 16

V = 1000000
ROWS_PER_W = 2 * D // NW
VMAIN = (V // CHUNK) * CHUNK
VTAIL = V - VMAIN
VP = VMAIN + CHUNK

CCHUNK = 16128
NCH = VMAIN // CCHUNK
NSLOT = 4
TOTCH = ROWS_PER_W * NCH


def _depad_kernel(uwt_h, iwt_h, uflat_h, iflat_h,
                  buf0_v, buf1_v, buf2_v, buf3_v, tail_v,
                  rsem, wsem, tsem):
    bufs = [buf0_v, buf1_v, buf2_v, buf3_v]
    wid = lax.axis_index("s") * NC + lax.axis_index("c")

    def src_dst(g):
        k = lax.div(g, NCH)
        c = lax.rem(g, NCH)
        r = wid * ROWS_PER_W + k
        f = lax.rem(r, D)
        return r, f, c * CCHUNK

    def fire_read(slot, g):
        r, f, off = src_dst(g)

        @pl.when(r < D)
        def _():
            pltpu.async_copy(uwt_h.at[f].at[pl.ds(off, CCHUNK)],
                             bufs[slot], rsem)

        @pl.when(r >= D)
        def _():
            pltpu.async_copy(iwt_h.at[f].at[pl.ds(off, CCHUNK)],
                             bufs[slot], rsem)

    def wait_read(slot, g):
        r, f, off = src_dst(g)

        @pl.when(r < D)
        def _():
            pltpu.make_async_copy(uwt_h.at[f].at[pl.ds(off, CCHUNK)],
                                  bufs[slot], rsem).wait()

        @pl.when(r >= D)
        def _():
            pltpu.make_async_copy(iwt_h.at[f].at[pl.ds(off, CCHUNK)],
                                  bufs[slot], rsem).wait()

    def fire_write(slot, g):
        r, f, off = src_dst(g)

        @pl.when(r < D)
        def _():
            pltpu.async_copy(bufs[slot],
                             uflat_h.at[pl.ds(f * VP + off, CCHUNK)], wsem)

        @pl.when(r >= D)
        def _():
            pltpu.async_copy(bufs[slot],
                             iflat_h.at[pl.ds(f * VP + off, CCHUNK)], wsem)

    def wait_write(slot, g):
        r, f, off = src_dst(g)

        @pl.when(r < D)
        def _():
            pltpu.make_async_copy(bufs[slot],
                                  uflat_h.at[pl.ds(f * VP + off, CCHUNK)],
                                  wsem).wait()

        @pl.when(r >= D)
        def _():
            pltpu.make_async_copy(bufs[slot],
                                  iflat_h.at[pl.ds(f * VP + off, CCHUNK)],
                                  wsem).wait()

    for slot in range(NSLOT):
        fire_read(slot, jnp.int32(slot))

    def round_body(t, carry):
        for slot in range(NSLOT):
            g = t * NSLOT + slot
            wait_read(slot, g)
            fire_write(slot, g)
        for slot in range(NSLOT):
            g = t * NSLOT + slot
            wait_write(slot, g)

            @pl.when(g + NSLOT < TOTCH)
            def _():
                fire_read(slot, g + NSLOT)

        return carry

    lax.fori_loop(0, TOTCH // NSLOT, round_body, 0)

    def tail_body(k, carry):
        r = wid * ROWS_PER_W + k
        f = lax.rem(r, D)

        @pl.when(r < D)
        def _():
            pltpu.async_copy(uwt_h.at[f].at[pl.ds(VMAIN, VTAIL)],
                             tail_v, tsem)
            pltpu.make_async_copy(uwt_h.at[f].at[pl.ds(VMAIN, VTAIL)],
                                  tail_v, tsem).wait()
            pltpu.async_copy(tail_v,
                             uflat_h.at[pl.ds(f * VP + VMAIN, VTAIL)], tsem)
            pltpu.make_async_copy(tail_v,
                                  uflat_h.at[pl.ds(f * VP + VMAIN, VTAIL)],
                                  tsem).wait()

        @pl.when(r >= D)
        def _():
            pltpu.async_copy(iwt_h.at[f].at[pl.ds(VMAIN, VTAIL)],
                             tail_v, tsem)
            pltpu.make_async_copy(iwt_h.at[f].at[pl.ds(VMAIN, VTAIL)],
                                  tail_v, tsem).wait()
            pltpu.async_copy(tail_v,
                             iflat_h.at[pl.ds(f * VP + VMAIN, VTAIL)], tsem)
            pltpu.make_async_copy(tail_v,
                                  iflat_h.at[pl.ds(f * VP + VMAIN, VTAIL)],
                                  tsem).wait()

        return carry

    lax.fori_loop(0, ROWS_PER_W, tail_body, 0)


def _fm_kernel(user_h, item_h, uwt_h, iwt_h, ubt_h, ibt_h, bias_h, out_h,
               uidx_v, iidx_v, ubuf_v, ibuf_v, ub_v, ib_v, bias_v, out_v,
               sem):
    wid = lax.axis_index("s") * NC + lax.axis_index("c")

    pltpu.sync_copy(user_h.at[wid], uidx_v)
    pltpu.sync_copy(item_h.at[wid], iidx_v)
    pltpu.sync_copy(bias_h, bias_v)

    for j in range(NCHUNK):
        pltpu.async_copy(ubt_h.at[0].at[uidx_v.at[j]],
                         ub_v.at[pl.ds(j * CHUNK, CHUNK)], sem)
        pltpu.async_copy(ibt_h.at[0].at[iidx_v.at[j]],
                         ib_v.at[pl.ds(j * CHUNK, CHUNK)], sem)

    def fire_f(f):
        for j in range(NCHUNK):
            pltpu.async_copy(uwt_h.at[f].at[uidx_v.at[j]],
                             ubuf_v.at[f, pl.ds(j * CHUNK, CHUNK)], sem)
            pltpu.async_copy(iwt_h.at[f].at[iidx_v.at[j]],
                             ibuf_v.at[f, pl.ds(j * CHUNK, CHUNK)], sem)

    def drain_f(f):
        for j in range(NCHUNK):
            pltpu.make_async_copy(uwt_h.at[f].at[uidx_v.at[j]],
                                  ubuf_v.at[f, pl.ds(j * CHUNK, CHUNK)],
                                  sem).wait()
            pltpu.make_async_copy(iwt_h.at[f].at[iidx_v.at[j]],
                                  ibuf_v.at[f, pl.ds(j * CHUNK, CHUNK)],
                                  sem).wait()

    def gather_f(f, carry):
        fire_f(f)

        @pl.when(f > 0)
        def _():
            drain_f(f - 1)

        return carry

    lax.fori_loop(0, D, gather_f, 0)
    drain_f(jnp.int32(D - 1))

    for j in range(NCHUNK):
        pltpu.make_async_copy(ubt_h.at[0].at[uidx_v.at[j]],
                              ub_v.at[pl.ds(j * CHUNK, CHUNK)], sem).wait()
        pltpu.make_async_copy(ibt_h.at[0].at[iidx_v.at[j]],
                              ib_v.at[pl.ds(j * CHUNK, CHUNK)], sem).wait()

    b0 = bias_v[...]

    def group_body(g, carry):
        acc = ub_v[pl.ds(g * 16, 16)] + ib_v[pl.ds(g * 16, 16)] + b0
        for f in range(D):
            gu = ubuf_v[f, pl.ds(g * 16, 16)]
            gi = ibuf_v[f, pl.ds(g * 16, 16)]
            acc = acc + gu * gi
        out_v[pl.ds(g * 16, 16)] = acc
        return carry

    lax.fori_loop(0, GROUPS, group_body, 0)
    pltpu.sync_copy(out_v, out_h.at[pl.ds(wid * BPW, BPW)])


def kernel(user, item, context, embed_user_w, embed_item_w,
           u_bias_w, i_bias_w, bias_):
    del context
    user3 = user.astype(jnp.int32).reshape(NW, NCHUNK, CHUNK)
    item3 = item.astype(jnp.int32).reshape(NW, NCHUNK, CHUNK)
    uw_t = embed_user_w.T
    iw_t = embed_item_w.T
    ub_t = u_bias_w.T
    ib_t = i_bias_w.T
    bias16 = jnp.broadcast_to(bias_.reshape(()), (16,))

    mesh = plsc.VectorSubcoreMesh(core_axis_name="c", subcore_axis_name="s")

    depad = functools.partial(
        pl.kernel,
        out_type=(jax.ShapeDtypeStruct((D * VP,), jnp.float32),
                  jax.ShapeDtypeStruct((D * VP,), jnp.float32)),
        mesh=mesh,
        compiler_params=pltpu.CompilerParams(
            needs_layout_passes=False, use_tc_tiling_on_sc=True),
        scratch_types=[
            pltpu.VMEM((CCHUNK,), jnp.float32),
            pltpu.VMEM((CCHUNK,), jnp.float32),
            pltpu.VMEM((CCHUNK,), jnp.float32),
            pltpu.VMEM((CCHUNK,), jnp.float32),
            pltpu.VMEM((VTAIL,), jnp.float32),
            pltpu.SemaphoreType.DMA,
            pltpu.SemaphoreType.DMA,
            pltpu.SemaphoreType.DMA,
        ],
    )(_depad_kernel)
    uflat, iflat = depad(uw_t, iw_t)

    fm = functools.partial(
        pl.kernel,
        out_type=jax.ShapeDtypeStruct((B,), jnp.float32),
        mesh=mesh,
        compiler_params=pltpu.CompilerParams(
            needs_layout_passes=False, use_tc_tiling_on_sc=False),
        scratch_types=[
            pltpu.VMEM((NCHUNK, CHUNK), jnp.int32),
            pltpu.VMEM((NCHUNK, CHUNK), jnp.int32),
            pltpu.VMEM((D, BPW), jnp.float32),
            pltpu.VMEM((D, BPW), jnp.float32),
            pltpu.VMEM((BPW,), jnp.float32),
            pltpu.VMEM((BPW,), jnp.float32),
            pltpu.VMEM((16,), jnp.float32),
            pltpu.VMEM((BPW,), jnp.float32),
            pltpu.SemaphoreType.DMA,
        ],
    )(_fm_kernel)
    return fm(user3, item3, uflat.reshape(D, VP), iflat.reshape(D, VP),
              ub_t, ib_t, bias16)
